# baseline (speedup 1.0000x reference)
import functools

import jax
import jax.numpy as jnp
from jax import lax
from jax.experimental import pallas as pl
from jax.experimental.pallas import tpu as pltpu
from jax.experimental.pallas import tpu_sc as plsc

B = 16384
D = 64
NC = 2
NS = 16
L = 16
NW = NC * NS
BPW = B // NW
C = 128
NCHUNK = BPW // C
G = C // L


def _body(heads, rels, tails, ent_h, ent_t, rel_w, rel_inv_w, out,
          idx_h, idx_r, idx_t, hh, rw, tt, ht, ri, th, out_v, sem):
    cid = lax.axis_index("c")
    sid = lax.axis_index("s")
    wid = sid * NC + cid
    base = wid * BPW

    for c in range(NCHUNK):
        off = base + c * C
        pltpu.sync_copy(heads.at[pl.ds(off, C)], idx_h.at[c])
        pltpu.sync_copy(rels.at[pl.ds(off, C)], idx_r.at[c])
        pltpu.sync_copy(tails.at[pl.ds(off, C)], idx_t.at[c])

    lane = lax.iota(jnp.int32, L)

    for c in range(NCHUNK):
        cps = [
            pltpu.async_copy(ent_h.at[idx_h.at[c]], hh, sem),
            pltpu.async_copy(rel_w.at[idx_r.at[c]], rw, sem),
            pltpu.async_copy(ent_t.at[idx_t.at[c]], tt, sem),
            pltpu.async_copy(ent_h.at[idx_t.at[c]], ht, sem),
            pltpu.async_copy(rel_inv_w.at[idx_r.at[c]], ri, sem),
            pltpu.async_copy(ent_t.at[idx_h.at[c]], th, sem),
        ]
        for cp in cps:
            cp.wait()

        for g in range(G):
            rowv = lane + (g * L)

            def dstep(d, acc, rowv=rowv):
                dv = jnp.broadcast_to(d, (L,))
                a = plsc.load_gather(hh, [rowv, dv])
                b = plsc.load_gather(rw, [rowv, dv])
                cc = plsc.load_gather(tt, [rowv, dv])
                p = plsc.load_gather(ht, [rowv, dv])
                q = plsc.load_gather(ri, [rowv, dv])
                r = plsc.load_gather(th, [rowv, dv])
                return acc + a * b * cc + p * q * r

            acc = lax.fori_loop(0, D, dstep, jnp.zeros((L,), jnp.float32))
            res = acc * 0.5
            res = jnp.minimum(jnp.maximum(res, -20.0), 20.0)
            out_v[pl.ds(c * C + g * L, L)] = res

    pltpu.sync_copy(out_v, out.at[pl.ds(base, BPW)])


@jax.jit
def kernel(heads, rels, tails, ent_h, ent_t, rel_w, rel_inv_w):
    mesh = plsc.VectorSubcoreMesh(
        core_axis_name="c", subcore_axis_name="s",
        num_cores=NC, num_subcores=NS)
    f = pl.kernel(
        _body,
        out_type=jax.ShapeDtypeStruct((B,), jnp.float32),
        mesh=mesh,
        compiler_params=pltpu.CompilerParams(
            needs_layout_passes=False, use_tc_tiling_on_sc=False),
        scratch_types=[
            pltpu.VMEM((NCHUNK, C), jnp.int32),
            pltpu.VMEM((NCHUNK, C), jnp.int32),
            pltpu.VMEM((NCHUNK, C), jnp.int32),
            pltpu.VMEM((C, D), jnp.float32),
            pltpu.VMEM((C, D), jnp.float32),
            pltpu.VMEM((C, D), jnp.float32),
            pltpu.VMEM((C, D), jnp.float32),
            pltpu.VMEM((C, D), jnp.float32),
            pltpu.VMEM((C, D), jnp.float32),
            pltpu.VMEM((BPW,), jnp.float32),
            pltpu.SemaphoreType.DMA,
        ],
    )
    return f(heads.astype(jnp.int32), rels.astype(jnp.int32),
             tails.astype(jnp.int32), ent_h, ent_t, rel_w, rel_inv_w)


# speedup vs baseline: 1.0232x; 1.0232x over previous
import jax
import jax.numpy as jnp
from jax import lax
from jax.experimental import pallas as pl
from jax.experimental.pallas import tpu as pltpu
from jax.experimental.pallas import tpu_sc as plsc

B = 16384
D = 64
NE = 1000000
NR = 1000
NC = 2
NS = 16
L = 16
NW = NC * NS
BPW = B // NW
C = 128
NCHUNK = BPW // C
G = C // L


def _body(heads, rels, tails, eh, et, rw, riw, out,
          idx_h, idx_r, idx_t, relv, b0, b1, qa, qb, fwd_v, out_v, sem):
    cid = lax.axis_index("c")
    sid = lax.axis_index("s")
    wid = sid * NC + cid
    base = wid * BPW

    pltpu.sync_copy(heads.at[pl.ds(base, BPW)], idx_h)
    pltpu.sync_copy(rels.at[pl.ds(base, BPW)], idx_r)
    pltpu.sync_copy(tails.at[pl.ds(base, BPW)], idx_t)

    lane = lax.iota(jnp.int32, L)

    def run_pass(rel_flat, ia_ref, ib_ref, emit):
        pltpu.sync_copy(rel_flat, relv)

        for c in range(NCHUNK):
            for g in range(G):
                va = ia_ref[pl.ds(c * C + g * L, L)]
                vb = ib_ref[pl.ds(c * C + g * L, L)]
                qa[pl.ds(g * L, L)] = va >> 1
                qb[pl.ds(g * L, L)] = vb >> 1
            cp0 = pltpu.async_copy(eh.at[qa], b0, sem)
            cp1 = pltpu.async_copy(et.at[qb], b1, sem)
            cp0.wait()
            cp1.wait()

            for g in range(G):
                va = ia_ref[pl.ds(c * C + g * L, L)]
                vb = ib_ref[pl.ds(c * C + g * L, L)]
                rowv = lane + g * L
                offa = (va & 1) * D
                offb = (vb & 1) * D
                q_vec = idx_r[pl.ds(c * C + g * L, L)]

                def dstep(d, acc, rowv=rowv, offa=offa, offb=offb,
                          q_vec=q_vec):
                    a = plsc.load_gather(b0, [rowv, offa + d])
                    b = plsc.load_gather(b1, [rowv, offb + d])
                    r = plsc.load_gather(relv, [q_vec + d * NR])
                    return acc + a * r * b

                acc = lax.fori_loop(0, D, dstep, jnp.zeros((L,), jnp.float32))
                emit(c * C + g * L, acc)

    def emit_fwd(off, acc):
        fwd_v[pl.ds(off, L)] = acc

    def emit_inv(off, acc):
        res = (fwd_v[pl.ds(off, L)] + acc) * 0.5
        res = jnp.minimum(jnp.maximum(res, -20.0), 20.0)
        out_v[pl.ds(off, L)] = res

    run_pass(rw, idx_h, idx_t, emit_fwd)
    run_pass(riw, idx_t, idx_h, emit_inv)

    pltpu.sync_copy(out_v, out.at[pl.ds(base, BPW)])


@jax.jit
def kernel(heads, rels, tails, ent_h, ent_t, rel_w, rel_inv_w):
    mesh = plsc.VectorSubcoreMesh(
        core_axis_name="c", subcore_axis_name="s",
        num_cores=NC, num_subcores=NS)
    f = pl.kernel(
        _body,
        out_type=jax.ShapeDtypeStruct((B,), jnp.float32),
        mesh=mesh,
        compiler_params=pltpu.CompilerParams(
            needs_layout_passes=False, use_tc_tiling_on_sc=True),
        scratch_types=[
            pltpu.VMEM((BPW,), jnp.int32),
            pltpu.VMEM((BPW,), jnp.int32),
            pltpu.VMEM((BPW,), jnp.int32),
            pltpu.VMEM((D * NR,), jnp.float32),
            pltpu.VMEM((C, 2 * D), jnp.float32),
            pltpu.VMEM((C, 2 * D), jnp.float32),
            pltpu.VMEM((C,), jnp.int32),
            pltpu.VMEM((C,), jnp.int32),
            pltpu.VMEM((BPW,), jnp.float32),
            pltpu.VMEM((BPW,), jnp.float32),
            pltpu.SemaphoreType.DMA,
        ],
    )
    return f(heads.astype(jnp.int32), rels.astype(jnp.int32),
             tails.astype(jnp.int32),
             ent_h.reshape(NE // 2, 2 * D), ent_t.reshape(NE // 2, 2 * D),
             rel_w.T.reshape(D * NR), rel_inv_w.T.reshape(D * NR))
